# Initial kernel scaffold; baseline (speedup 1.0000x reference)
import jax
import jax.numpy as jnp
from jax.experimental import pallas as pl


def kernel(X, W, b):
    raise NotImplementedError("write your pallas kernel here")


# speedup vs baseline: 1.6096x; 1.6096x over previous
import functools
import math

import jax
import jax.numpy as jnp
from jax import lax
from jax.experimental import pallas as pl
from jax.experimental.pallas import tpu as pltpu
from jax.experimental.pallas import tpu_sc as plsc

_NUM_VARIABLES = 8
_D_MODEL = 128
_DPV = _D_MODEL // _NUM_VARIABLES
_SCALE = math.sqrt(_D_MODEL / _DPV)

_CHUNK = 256


def _sc_body(n_tokens, n_workers, x_hbm, w_hbm, b_hbm, out_hbm,
             x_v, out_v, w_v, b_v):
    info = plsc.get_sparse_core_info()
    nc = info.num_cores
    wid = lax.axis_index("s") * nc + lax.axis_index("c")
    per_w = n_tokens // n_workers
    n_chunks = per_w // _CHUNK
    w_base = wid * per_w

    iota = lax.iota(jnp.int32, 16)
    zeros = jnp.zeros((16,), jnp.float32)

    pltpu.sync_copy(w_hbm, w_v)
    pltpu.sync_copy(b_hbm, b_v)
    for j in range(_D_MODEL // 16):
        w_v[pl.ds(j * 16, 16)] = w_v[pl.ds(j * 16, 16)] * _SCALE
        b_v[pl.ds(j * 16, 16)] = b_v[pl.ds(j * 16, 16)] * _SCALE

    def chunk_body(g, _):
        t_base = w_base + g * _CHUNK
        pltpu.sync_copy(x_hbm.at[pl.ds(t_base * 2, _CHUNK * 2)], x_v)

        def zero_body(r, _):
            for j in range(8):
                out_v[pl.ds(r * 128 + j * 16, 16)] = zeros
            return 0
        lax.fori_loop(0, _CHUNK, zero_body, 0, unroll=4)

        def tile_body(i, _):
            t0 = i * 16
            tok2 = (t0 + iota) * 2
            vals = plsc.load_gather(x_v, [tok2])
            u = plsc.load_gather(x_v, [tok2 + 1]).astype(jnp.int32)
            u16 = u * 16
            base = (t0 + iota) * 128 + u16
            for l in range(16):
                wg = plsc.load_gather(w_v, [u16 + l])
                bg = plsc.load_gather(b_v, [u16 + l])
                plsc.store_scatter(out_v, [base + l], vals * wg + bg)
            return 0
        lax.fori_loop(0, _CHUNK // 16, tile_body, 0)

        pltpu.sync_copy(out_v, out_hbm.at[pl.ds(t_base * 128, _CHUNK * 128)])
        return 0

    lax.fori_loop(0, n_chunks, chunk_body, 0)


def kernel(X, W, b):
    B, S, _ = X.shape
    n_tokens = B * S
    info = plsc.get_sparse_core_info()
    n_workers = info.num_cores * info.num_subcores

    x_flat = X.reshape(n_tokens * 2)
    w_flat = W.reshape(_D_MODEL)

    mesh = plsc.VectorSubcoreMesh(core_axis_name="c", subcore_axis_name="s")
    body = functools.partial(_sc_body, n_tokens, n_workers)
    out = pl.kernel(
        body,
        out_type=jax.ShapeDtypeStruct((n_tokens * _D_MODEL,), jnp.float32),
        mesh=mesh,
        compiler_params=pltpu.CompilerParams(needs_layout_passes=False),
        scratch_types=[
            pltpu.VMEM((_CHUNK * 2,), jnp.float32),
            pltpu.VMEM((_CHUNK * _D_MODEL,), jnp.float32),
            pltpu.VMEM((_D_MODEL,), jnp.float32),
            pltpu.VMEM((_D_MODEL,), jnp.float32),
        ],
    )(x_flat, w_flat, b)
    return out.reshape(B, S, _D_MODEL)


# speedup vs baseline: 1.7955x; 1.1155x over previous
import functools
import math

import jax
import jax.numpy as jnp
from jax import lax
from jax.experimental import pallas as pl
from jax.experimental.pallas import tpu as pltpu
from jax.experimental.pallas import tpu_sc as plsc

_NUM_VARIABLES = 8
_D_MODEL = 128
_DPV = _D_MODEL // _NUM_VARIABLES
_SCALE = math.sqrt(_D_MODEL / _DPV)

_CHUNK = 400
_OUT_W = _CHUNK * _D_MODEL


def _sc_body(n_tokens, n_workers, x_hbm, w_hbm, b_hbm, out_hbm,
             x_v, out_v, w_v, b_v, xs0, xs1, os0, os1):
    info = plsc.get_sparse_core_info()
    nc = info.num_cores
    wid = lax.axis_index("s") * nc + lax.axis_index("c")
    per_w = n_tokens // n_workers
    n_chunks = per_w // _CHUNK
    w_base = wid * per_w

    iota = lax.iota(jnp.int32, 16)
    zeros = jnp.zeros((16,), jnp.float32)
    x_sems = (xs0, xs1)
    o_sems = (os0, os1)

    pltpu.sync_copy(w_hbm, w_v)
    pltpu.sync_copy(b_hbm, b_v)
    for j in range(_D_MODEL // 16):
        w_v[pl.ds(j * 16, 16)] = w_v[pl.ds(j * 16, 16)] * _SCALE
        b_v[pl.ds(j * 16, 16)] = b_v[pl.ds(j * 16, 16)] * _SCALE

    def x_copy(g, p):
        return pltpu.make_async_copy(
            x_hbm.at[pl.ds((w_base + g * _CHUNK) * 2, _CHUNK * 2)],
            x_v.at[pl.ds(p * _CHUNK * 2, _CHUNK * 2)],
            x_sems[p])

    def o_copy(g, p):
        return pltpu.make_async_copy(
            out_v.at[pl.ds(p * _OUT_W, _OUT_W)],
            out_hbm.at[pl.ds((w_base + g * _CHUNK) * 128, _OUT_W)],
            o_sems[p])

    x_copy(0, 0).start()
    x_copy(1, 1).start()

    def chunk_pair(g2, _):
        for p in range(2):
            g = 2 * g2 + p
            x_copy(g, p).wait()
            @pl.when(g2 >= 1)
            def _():
                o_copy(g, p).wait()

            xo = p * _CHUNK * 2
            oo = p * _OUT_W

            def zero_body(r, _):
                base = oo + r * 128
                for j in range(8):
                    out_v[pl.ds(base + j * 16, 16)] = zeros
                return 0
            lax.fori_loop(0, _CHUNK, zero_body, 0, unroll=4)

            def tile_body(i, _):
                t0 = i * 16
                tok2 = xo + (t0 + iota) * 2
                vals = plsc.load_gather(x_v, [tok2])
                u = plsc.load_gather(x_v, [tok2 + 1]).astype(jnp.int32)
                u16 = u * 16
                base = oo + (t0 + iota) * 128 + u16
                for l in range(16):
                    wg = plsc.load_gather(w_v, [u16 + l])
                    bg = plsc.load_gather(b_v, [u16 + l])
                    plsc.store_scatter(out_v, [base + l], vals * wg + bg)
                return 0
            lax.fori_loop(0, _CHUNK // 16, tile_body, 0)

            o_copy(g, p).start()
            @pl.when(g2 <= n_chunks // 2 - 2)
            def _():
                x_copy(g + 2, p).start()
        return 0

    lax.fori_loop(0, n_chunks // 2, chunk_pair, 0)

    o_copy(n_chunks - 2, 0).wait()
    o_copy(n_chunks - 1, 1).wait()


def kernel(X, W, b):
    B, S, _ = X.shape
    n_tokens = B * S
    info = plsc.get_sparse_core_info()
    n_workers = info.num_cores * info.num_subcores

    x_flat = X.reshape(n_tokens * 2)
    w_flat = W.reshape(_D_MODEL)

    mesh = plsc.VectorSubcoreMesh(core_axis_name="c", subcore_axis_name="s")
    body = functools.partial(_sc_body, n_tokens, n_workers)
    out = pl.kernel(
        body,
        out_type=jax.ShapeDtypeStruct((n_tokens * _D_MODEL,), jnp.float32),
        mesh=mesh,
        compiler_params=pltpu.CompilerParams(needs_layout_passes=False),
        scratch_types=[
            pltpu.VMEM((2 * _CHUNK * 2,), jnp.float32),
            pltpu.VMEM((2 * _OUT_W,), jnp.float32),
            pltpu.VMEM((_D_MODEL,), jnp.float32),
            pltpu.VMEM((_D_MODEL,), jnp.float32),
            pltpu.SemaphoreType.DMA,
            pltpu.SemaphoreType.DMA,
            pltpu.SemaphoreType.DMA,
            pltpu.SemaphoreType.DMA,
        ],
    )(x_flat, w_flat, b)
    return out.reshape(B, S, _D_MODEL)
